# baseline (speedup 1.0000x reference)
import numpy as np

import jax
import jax.numpy as jnp
from jax.experimental import pallas as pl
from jax.experimental.pallas import tpu as pltpu
from jax.sharding import Mesh, PartitionSpec as P

_KS = 3
_ST = 2
_H = 512
_W = 512
_HO = (_H - _KS) // _ST + 1
_WO = (_W - _KS) // _ST + 1
_LC = 128
_NC = _W // _LC
_CG = 16


def _pool_body(x0_ref, x1_ref, x2_ref, x3_ref, t_ref, o_ref):
    slabs = (x0_ref, x1_ref, x2_ref, x3_ref)
    for g in range(_CG):
        acc = None
        for ci, xc in enumerate(slabs):
            a = xc[:, pl.ds(g, 1), pl.ds(0, _HO, _ST), :]
            b = xc[:, pl.ds(g, 1), pl.ds(1, _HO, _ST), :]
            d = xc[:, pl.ds(g, 1), pl.ds(2, _HO, _ST), :]
            rows = (a + b + d)[0, 0].astype(jnp.bfloat16)
            part = jnp.dot(rows, t_ref[ci],
                           preferred_element_type=jnp.float32)
            acc = part if acc is None else acc + part
        o_ref[0, g] = acc[:, :_WO]


def _colpool_matrix():
    k = jnp.arange(_W, dtype=jnp.int32)[:, None]
    j = jnp.arange(256, dtype=jnp.int32)[None, :]
    d = k - _ST * j
    t = jnp.where((d >= 0) & (d < _KS), 1.0 / (_KS * _KS), 0.0)
    return t.astype(jnp.bfloat16).reshape(_NC, _LC, 256)


def _pool_one_device(x):
    bsz, ch, h, w = x.shape
    tmat = _colpool_matrix()

    def _x_spec(ci):
        return pl.BlockSpec((1, _CG, _H, _LC), lambda b, g: (b, g, 0, ci))

    return pl.pallas_call(
        _pool_body,
        grid=(bsz, ch // _CG),
        in_specs=[_x_spec(0), _x_spec(1), _x_spec(2), _x_spec(3),
                  pl.BlockSpec((_NC, _LC, 256), lambda b, g: (0, 0, 0))],
        out_specs=pl.BlockSpec((1, _CG, _HO, _WO), lambda b, g: (b, g, 0, 0)),
        out_shape=jax.ShapeDtypeStruct((bsz, ch, _HO, _WO), x.dtype),
        compiler_params=pltpu.CompilerParams(
            dimension_semantics=("parallel", "arbitrary"),
        ),
    )(x, x, x, x, tmat)


def kernel(x):
    devs = [d for d in jax.devices() if d.platform == "tpu"]
    ndev = 2 if len(devs) >= 2 and x.shape[0] % 2 == 0 else 1
    if ndev == 1:
        return _pool_one_device(x)
    mesh = Mesh(np.array(devs[:ndev]), ("d",))
    f = jax.shard_map(_pool_one_device, mesh=mesh,
                      in_specs=P("d"), out_specs=P("d"), check_vma=False)
    return f(x)


# speedup vs baseline: 3.0027x; 3.0027x over previous
import jax
import jax.numpy as jnp
from jax.experimental import pallas as pl
from jax.experimental.pallas import tpu as pltpu
_KS = 3
_ST = 2
_H = 512
_W = 512
_HO = (_H - _KS) // _ST + 1
_WO = (_W - _KS) // _ST + 1
_LC = 128
_NC = _W // _LC
_CG = 16


def _pool_body(x0_ref, x1_ref, x2_ref, x3_ref, t_ref, o_ref):
    slabs = (x0_ref, x1_ref, x2_ref, x3_ref)
    for g in range(_CG):
        acc = None
        for ci, xc in enumerate(slabs):
            a = xc[:, pl.ds(g, 1), pl.ds(0, _HO, _ST), :]
            b = xc[:, pl.ds(g, 1), pl.ds(1, _HO, _ST), :]
            d = xc[:, pl.ds(g, 1), pl.ds(2, _HO, _ST), :]
            rows = (a + b + d)[0, 0].astype(jnp.bfloat16)
            part = jnp.dot(rows, t_ref[ci],
                           preferred_element_type=jnp.float32)
            acc = part if acc is None else acc + part
        o_ref[0, g] = acc[:, :_WO]


def _colpool_matrix():
    k = jnp.arange(_W, dtype=jnp.int32)[:, None]
    j = jnp.arange(256, dtype=jnp.int32)[None, :]
    d = k - _ST * j
    t = jnp.where((d >= 0) & (d < _KS), 1.0 / (_KS * _KS), 0.0)
    return t.astype(jnp.bfloat16).reshape(_NC, _LC, 256)


def _pool_one_device(x):
    bsz, ch, h, w = x.shape
    tmat = _colpool_matrix()

    def _x_spec(ci):
        return pl.BlockSpec((1, _CG, _H, _LC), lambda b, g: (b, g, 0, ci))

    return pl.pallas_call(
        _pool_body,
        grid=(bsz, ch // _CG),
        in_specs=[_x_spec(0), _x_spec(1), _x_spec(2), _x_spec(3),
                  pl.BlockSpec((_NC, _LC, 256), lambda b, g: (0, 0, 0))],
        out_specs=pl.BlockSpec((1, _CG, _HO, _WO), lambda b, g: (b, g, 0, 0)),
        out_shape=jax.ShapeDtypeStruct((bsz, ch, _HO, _WO), x.dtype),
        compiler_params=pltpu.CompilerParams(
            dimension_semantics=("parallel", "arbitrary"),
        ),
    )(x, x, x, x, tmat)


def kernel(x):
    return _pool_one_device(x)
